# Initial kernel scaffold; baseline (speedup 1.0000x reference)
import jax
import jax.numpy as jnp
from jax.experimental import pallas as pl


def kernel(x, edge_index, batch, bn_in_g, bn_in_b, W0, b0, g0, be0, W1, b1, g1, be1, W2, b2, g2, be2, cW1, cb1, cW2, cb2):
    raise NotImplementedError("write your pallas kernel here")


# speedup vs baseline: 11.4053x; 11.4053x over previous
import functools

import jax
import jax.numpy as jnp
from jax import lax
from jax.experimental import pallas as pl
from jax.experimental.pallas import tpu as pltpu
from jax.experimental.pallas import tpu_sc as plsc

N = 10000
E = 320000
D_IN = 128
D_H = 64
N_CLASSES = 2
N_GRAPHS = 16
EPS = 1e-5

NC = 2
NS = 16
CHUNK = 128
N_CHUNKS = -(-E // (NC * NS * CHUNK))
E_PAD = NC * NS * N_CHUNKS * CHUNK
ROWS_PER_TILE = 640
N_PAD = NS * ROWS_PER_TILE

@functools.cache
def _sc_kernels():
    mesh = plsc.VectorSubcoreMesh(core_axis_name="c", subcore_axis_name="s",
                                  num_cores=NC, num_subcores=NS)

    @functools.partial(
        pl.kernel,
        out_type=jax.ShapeDtypeStruct((NC, N_PAD, 16), jnp.float32),
        mesh=mesh,
        scratch_types=[
            pltpu.VMEM_SHARED((N_PAD, 16), jnp.float32),
            pltpu.VMEM((CHUNK,), jnp.int32),
            pltpu.VMEM((CHUNK, 16), jnp.float32),
        ],
        compiler_params=pltpu.CompilerParams(use_tc_tiling_on_sc=False),
    )
    def sc_degree(dst_hbm, zeros_hbm, ones_hbm, out_hbm, acc, idx_v, ones_v):
        c = lax.axis_index("c")
        s = lax.axis_index("s")
        row0 = pl.multiple_of(s * ROWS_PER_TILE, 8)
        pltpu.sync_copy(zeros_hbm.at[pl.ds(row0, ROWS_PER_TILE)],
                        acc.at[pl.ds(row0, ROWS_PER_TILE)])
        pltpu.sync_copy(ones_hbm, ones_v)
        plsc.subcore_barrier()

        def body(j, carry):
            pltpu.sync_copy(dst_hbm.at[c, s, j], idx_v)
            pltpu.sync_copy(ones_v, acc.at[idx_v], add=True)
            return carry

        lax.fori_loop(0, N_CHUNKS, body, 0)
        plsc.subcore_barrier()
        pltpu.sync_copy(acc.at[pl.ds(row0, ROWS_PER_TILE)],
                        out_hbm.at[c, pl.ds(row0, ROWS_PER_TILE)])

    @functools.partial(
        pl.kernel,
        out_type=jax.ShapeDtypeStruct((NC, N_PAD, D_H), jnp.float32),
        mesh=mesh,
        scratch_types=[
            pltpu.VMEM_SHARED((N_PAD, D_H), jnp.float32),
            pltpu.VMEM((CHUNK,), jnp.int32),
            pltpu.VMEM((CHUNK,), jnp.int32),
            pltpu.VMEM((CHUNK, D_H), jnp.float32),
            pltpu.SemaphoreType.DMA,
        ],
        compiler_params=pltpu.CompilerParams(use_tc_tiling_on_sc=False),
    )
    def sc_scatter(g_hbm, src_hbm, dst_hbm, zeros_hbm, out_hbm,
                   acc, idxg_v, idxs_v, rows_v, sem):
        c = lax.axis_index("c")
        s = lax.axis_index("s")
        row0 = pl.multiple_of(s * ROWS_PER_TILE, 8)
        pltpu.sync_copy(zeros_hbm.at[pl.ds(row0, ROWS_PER_TILE)],
                        acc.at[pl.ds(row0, ROWS_PER_TILE)])
        plsc.subcore_barrier()

        def body(j, carry):
            pltpu.sync_copy(src_hbm.at[c, s, j], idxg_v)
            pltpu.sync_copy(dst_hbm.at[c, s, j], idxs_v)
            pltpu.async_copy(g_hbm.at[idxg_v], rows_v, sem).wait()
            pltpu.sync_copy(rows_v, acc.at[idxs_v], add=True)
            return carry

        lax.fori_loop(0, N_CHUNKS, body, 0)
        plsc.subcore_barrier()
        pltpu.sync_copy(acc.at[pl.ds(row0, ROWS_PER_TILE)],
                        out_hbm.at[c, pl.ds(row0, ROWS_PER_TILE)])

    return sc_degree, sc_scatter


def _dot_t(a, w):
    return lax.dot_general(a, w, (((1,), (1,)), ((), ())),
                           precision=lax.Precision.HIGHEST,
                           preferred_element_type=jnp.float32)


def _bn_rows(h, gamma, beta):
    mu = jnp.mean(h, axis=0)
    var = jnp.mean((h - mu) ** 2, axis=0)
    return (h - mu) * lax.rsqrt(var + EPS) * gamma + beta


def _tc_prep_body(x_ref, bng_ref, bnb_ref, w0_ref, degp_ref,
                  g_ref, dinv_ref):
    x = x_ref[...]
    h = _bn_rows(x, bng_ref[...], bnb_ref[...])
    h1 = _dot_t(h, w0_ref[...])
    deg = degp_ref[0, :, 0:1] + degp_ref[1, :, 0:1] + 1.0
    dinv = lax.rsqrt(deg)
    g_ref[...] = jnp.concatenate(
        [h1 * dinv[:N], jnp.zeros((N_PAD - N, D_H), jnp.float32)], axis=0)
    dinv_ref[...] = dinv


_tc_prep = pl.pallas_call(
    _tc_prep_body,
    out_shape=(jax.ShapeDtypeStruct((N_PAD, D_H), jnp.float32),
               jax.ShapeDtypeStruct((N_PAD, 1), jnp.float32)),
)


def _tc_mid_body(p_ref, gprev_ref, dinv_ref, b_ref, gam_ref, bet_ref, w_ref,
                 g_ref):
    dinv = dinv_ref[...]
    acc = p_ref[0, :N, :] + p_ref[1, :N, :] + gprev_ref[:N, :]
    out = acc * dinv[:N] + b_ref[...]
    h = jnp.maximum(_bn_rows(out, gam_ref[...], bet_ref[...]), 0.0)
    h1 = _dot_t(h, w_ref[...])
    g_ref[...] = jnp.concatenate(
        [h1 * dinv[:N], jnp.zeros((N_PAD - N, D_H), jnp.float32)], axis=0)


_tc_mid = pl.pallas_call(
    _tc_mid_body,
    out_shape=jax.ShapeDtypeStruct((N_PAD, D_H), jnp.float32),
)


def _tc_final_body(p_ref, gprev_ref, dinv_ref, b_ref, gam_ref, bet_ref,
                   batch_ref, cw1_ref, cb1_ref, cw2_ref, cb2_ref, res_ref):
    dinv = dinv_ref[...]
    acc = p_ref[0, :N, :] + p_ref[1, :N, :] + gprev_ref[:N, :]
    out = acc * dinv[:N] + b_ref[...]
    h = jnp.maximum(_bn_rows(out, gam_ref[...], bet_ref[...]), 0.0)
    seg = batch_ref[...]
    oh = (seg == lax.broadcasted_iota(jnp.int32, (1, N_GRAPHS), 1))
    oh = oh.astype(jnp.float32)
    sums = lax.dot_general(oh, h, (((0,), (0,)), ((), ())),
                           precision=lax.Precision.HIGHEST,
                           preferred_element_type=jnp.float32)
    cnt = jnp.sum(oh, axis=0)[:, None]
    pooled = sums / jnp.maximum(cnt, 1.0)
    hc = jnp.maximum(_dot_t(pooled, cw1_ref[...]) + cb1_ref[...], 0.0)
    res_ref[...] = _dot_t(hc, cw2_ref[...]) + cb2_ref[...]


_tc_final = pl.pallas_call(
    _tc_final_body,
    out_shape=jax.ShapeDtypeStruct((N_GRAPHS, N_CLASSES), jnp.float32),
)


def kernel(x, edge_index, batch, bn_in_g, bn_in_b, W0, b0, g0, be0,
           W1, b1, g1, be1, W2, b2, g2, be2, cW1, cb1, cW2, cb2):
    pad = E_PAD - E
    src = jnp.concatenate([edge_index[0], jnp.full((pad,), N, jnp.int32)])
    dst = jnp.concatenate([edge_index[1], jnp.full((pad,), N, jnp.int32)])
    src = src.reshape(NC, NS, N_CHUNKS, CHUNK)
    dst = dst.reshape(NC, NS, N_CHUNKS, CHUNK)

    zeros16 = jnp.zeros((N_PAD, 16), jnp.float32)
    ones16 = jnp.ones((CHUNK, 16), jnp.float32)
    zeros64 = jnp.zeros((N_PAD, D_H), jnp.float32)
    batch2d = batch.reshape(N, 1)

    sc_degree, sc_scatter = _sc_kernels()
    degp = sc_degree(dst, zeros16, ones16)
    gfeat, dinv = _tc_prep(x, bn_in_g, bn_in_b, W0, degp)

    p = sc_scatter(gfeat, src, dst, zeros64)
    gfeat = _tc_mid(p, gfeat, dinv, b0, g0, be0, W1)

    p = sc_scatter(gfeat, src, dst, zeros64)
    gfeat = _tc_mid(p, gfeat, dinv, b1, g1, be1, W2)

    p = sc_scatter(gfeat, src, dst, zeros64)
    return _tc_final(p, gfeat, dinv, b2, g2, be2, batch2d, cW1, cb1, cW2, cb2)
